# Initial kernel scaffold; baseline (speedup 1.0000x reference)
import jax
import jax.numpy as jnp
from jax.experimental import pallas as pl


def kernel(pts, cb0, cb1, cb2, cb3, cb4, cb5, cb6, cb7):
    raise NotImplementedError("write your pallas kernel here")


# speedup vs baseline: 10.4616x; 10.4616x over previous
import functools
import math

import jax
import jax.numpy as jnp
from jax import lax
from jax.experimental import pallas as pl
from jax.experimental.pallas import tpu as pltpu
from jax.experimental.pallas import tpu_sc as plsc

_BASE_RES = 16
_MAX_RES = 256
_NUM_LOD = 8
_FEAT = 2
_N = 1048576
_GROWTH = math.exp((math.log(_MAX_RES) - math.log(_BASE_RES)) / (_NUM_LOD - 1))
_LODS = [int(_BASE_RES * _GROWTH ** L) for L in range(_NUM_LOD)]

_NC = 2
_NS = 16
_NW = _NC * _NS
_PPW = _N // _NW
_C = 1024
_CHUNKS = _PPW // _C


def _make_lookup():
    mesh = plsc.VectorSubcoreMesh(
        core_axis_name="c", subcore_axis_name="s",
        num_cores=_NC, num_subcores=_NS)

    @functools.partial(
        pl.kernel,
        out_type=jax.ShapeDtypeStruct((_N * _NUM_LOD * _FEAT,), jnp.float32),
        mesh=mesh,
        compiler_params=pltpu.CompilerParams(
            needs_layout_passes=False, use_tc_tiling_on_sc=False),
        scratch_types=[
            pltpu.VMEM((_C,), jnp.float32),
            pltpu.VMEM((_C,), jnp.float32),
            pltpu.VMEM((_NUM_LOD, _C), jnp.int32),
            pltpu.VMEM((_NUM_LOD * _FEAT, _C), jnp.float32),
            pltpu.VMEM((_C * 16,), jnp.float32),
            pltpu.SemaphoreType.DMA,
        ],
    )
    def lookup(xs_h, ys_h, *rest):
        cbs = rest[:_NUM_LOD * _FEAT]
        out_h = rest[_NUM_LOD * _FEAT]
        xv, yv, idxv, colv, outv, sem = rest[_NUM_LOD * _FEAT + 1:]
        wid = lax.axis_index("s") * _NC + lax.axis_index("c")
        iota = lax.iota(jnp.int32, 16)
        oconsts = [iota * 16 + j for j in range(16)]

        def chunk_body(ci, carry):
            base = pl.multiple_of(wid * _PPW + ci * _C, _C)
            pltpu.sync_copy(xs_h.at[pl.ds(base, _C)], xv)
            pltpu.sync_copy(ys_h.at[pl.ds(base, _C)], yv)

            def idx_body(j, c2):
                x = xv[pl.ds(j * 16, 16)]
                y = yv[pl.ds(j * 16, 16)]
                for l, r in enumerate(_LODS):
                    xi = (x * (r - 1.0)).astype(jnp.int32)
                    yi = (y * (r - 1.0)).astype(jnp.int32)
                    idxv[l, pl.ds(j * 16, 16)] = xi + yi * r
                return c2
            lax.fori_loop(0, _C // 16, idx_body, 0)

            copies = []
            for l in range(_NUM_LOD):
                for f in range(_FEAT):
                    copies.append(pltpu.async_copy(
                        cbs[f * _NUM_LOD + l].at[idxv.at[l]],
                        colv.at[f * _NUM_LOD + l], sem))
            for cpy in copies:
                cpy.wait()

            def asm_body(i, c2):
                n16 = i * 256
                for j in range(16):
                    v = colv[j, pl.ds(i * 16, 16)]
                    plsc.store_scatter(outv, [oconsts[j] + n16], v)
                return c2
            lax.fori_loop(0, _C // 16, asm_body, 0)

            pltpu.sync_copy(outv, out_h.at[pl.ds(base * 16, _C * 16)])
            return carry

        lax.fori_loop(0, _CHUNKS, chunk_body, 0)

    return lookup


_lookup = _make_lookup()


def kernel(pts, cb0, cb1, cb2, cb3, cb4, cb5, cb6, cb7):
    xs = jnp.ravel(pts[:, 0])
    ys = jnp.ravel(pts[:, 1])
    cbs = [cb0, cb1, cb2, cb3, cb4, cb5, cb6, cb7]
    flat = ([jnp.ravel(cb[:, 0]) for cb in cbs]
            + [jnp.ravel(cb[:, 1]) for cb in cbs])
    out = _lookup(xs, ys, *flat)
    return out.reshape(_N, _NUM_LOD * _FEAT)
